# baseline (speedup 1.0000x reference)
import functools

import jax
import jax.numpy as jnp
from jax import lax
from jax.experimental import pallas as pl
from jax.experimental.pallas import tpu as pltpu
from jax.experimental.pallas import tpu_sc as plsc

_CHUNK = 256


def _gather_call(table_wide, idx, b_per_w, num_cores):
    B = idx.shape[0]
    D = table_wide.shape[1] // 2
    mesh = plsc.VectorSubcoreMesh(core_axis_name="c", subcore_axis_name="s")

    @functools.partial(
        pl.kernel,
        mesh=mesh,
        out_type=jax.ShapeDtypeStruct((B, D), jnp.float32),
        scratch_types=[
            pltpu.VMEM((b_per_w,), jnp.int32),
            pltpu.VMEM((_CHUNK,), jnp.int32),
            pltpu.VMEM((_CHUNK, 2 * D), jnp.float32),
            pltpu.VMEM((_CHUNK, D), jnp.float32),
            pltpu.SemaphoreType.DMA,
        ],
    )
    def gather_kernel(
        table_hbm, idx_hbm, out_hbm, idx_v, q_v, wide_v, rows_v, sem
    ):
        wid = lax.axis_index("s") * num_cores + lax.axis_index("c")
        base = wid * b_per_w
        pltpu.sync_copy(idx_hbm.at[pl.ds(base, b_per_w)], idx_v)

        for c in range(b_per_w // _CHUNK):
            off = c * _CHUNK

            def pair_ids(g, carry, off=off):
                vec = idx_v[pl.ds(off + g * 16, 16)]
                q_v[pl.ds(g * 16, 16)] = lax.shift_right_logical(vec, 1)
                return carry

            lax.fori_loop(0, _CHUNK // 16, pair_ids, 0)
            pltpu.async_copy(table_hbm.at[q_v], wide_v, sem).wait()

            def select(g, carry, off=off):
                vec = idx_v[pl.ds(off + g * 16, 16)]
                off_vec = lax.mul(lax.bitwise_and(vec, 1), D)
                for lane in range(16):
                    j = g * 16 + lane
                    half = off_vec[lane]
                    for k in range(D // 16):
                        rows_v[j, pl.ds(k * 16, 16)] = wide_v[
                            j, pl.ds(half + k * 16, 16)
                        ]
                return carry

            lax.fori_loop(0, _CHUNK // 16, select, 0)
            pltpu.sync_copy(rows_v, out_hbm.at[pl.ds(base + off, _CHUNK)])

    return gather_kernel(table_wide, idx)


def kernel(expression_face, rand_id):
    info = plsc.get_sparse_core_info()
    nw = info.num_cores * info.num_subcores
    B = rand_id.shape[0]
    b_per_w = B // nw
    table_wide = jnp.concatenate(
        [expression_face[0::2], expression_face[1::2]], axis=1
    )
    return _gather_call(
        table_wide, rand_id.astype(jnp.int32), b_per_w, info.num_cores
    )


# speedup vs baseline: 8.9734x; 8.9734x over previous
import functools

import jax
import jax.numpy as jnp
from jax import lax
from jax.experimental import pallas as pl
from jax.experimental.pallas import tpu as pltpu
from jax.experimental.pallas import tpu_sc as plsc

_CHUNK = 256
_RPB = 1600


def _repack(table):
    V, D = table.shape

    def repack_kernel(in_ref, out_ref):
        a = in_ref[...].reshape(_RPB // 2, 2, D)
        out_ref[:, 0:D] = a[:, 0, :]
        out_ref[:, D : 2 * D] = a[:, 1, :]

    return pl.pallas_call(
        repack_kernel,
        grid=(V // _RPB,),
        in_specs=[pl.BlockSpec((_RPB, D), lambda i: (i, 0))],
        out_specs=pl.BlockSpec((_RPB // 2, 2 * D), lambda i: (i, 0)),
        out_shape=jax.ShapeDtypeStruct((V // 2, 2 * D), jnp.float32),
    )(table)


def _gather_call(table_wide, idx, b_per_w, num_cores):
    B = idx.shape[0]
    D = table_wide.shape[1] // 2
    mesh = plsc.VectorSubcoreMesh(core_axis_name="c", subcore_axis_name="s")

    @functools.partial(
        pl.kernel,
        mesh=mesh,
        out_type=jax.ShapeDtypeStruct((B, D), jnp.float32),
        scratch_types=[
            pltpu.VMEM((b_per_w,), jnp.int32),
            pltpu.VMEM((_CHUNK,), jnp.int32),
            pltpu.VMEM((_CHUNK, 2 * D), jnp.float32),
            pltpu.VMEM((_CHUNK, D), jnp.float32),
            pltpu.SemaphoreType.DMA,
        ],
    )
    def gather_kernel(
        table_hbm, idx_hbm, out_hbm, idx_v, q_v, wide_v, rows_v, sem
    ):
        wid = lax.axis_index("s") * num_cores + lax.axis_index("c")
        base = wid * b_per_w
        pltpu.sync_copy(idx_hbm.at[pl.ds(base, b_per_w)], idx_v)

        for c in range(b_per_w // _CHUNK):
            off = c * _CHUNK

            def pair_ids(g, carry, off=off):
                vec = idx_v[pl.ds(off + g * 16, 16)]
                q_v[pl.ds(g * 16, 16)] = lax.shift_right_logical(vec, 1)
                return carry

            lax.fori_loop(0, _CHUNK // 16, pair_ids, 0)
            pltpu.async_copy(table_hbm.at[q_v], wide_v, sem).wait()

            def select(g, carry, off=off):
                vec = idx_v[pl.ds(off + g * 16, 16)]
                off_vec = lax.mul(lax.bitwise_and(vec, 1), D)
                for lane in range(16):
                    j = g * 16 + lane
                    half = off_vec[lane]
                    for k in range(D // 16):
                        rows_v[j, pl.ds(k * 16, 16)] = wide_v[
                            j, pl.ds(half + k * 16, 16)
                        ]
                return carry

            lax.fori_loop(0, _CHUNK // 16, select, 0)
            pltpu.sync_copy(rows_v, out_hbm.at[pl.ds(base + off, _CHUNK)])

    return gather_kernel(table_wide, idx)


def kernel(expression_face, rand_id):
    info = plsc.get_sparse_core_info()
    nw = info.num_cores * info.num_subcores
    B = rand_id.shape[0]
    b_per_w = B // nw
    table_wide = _repack(expression_face)
    return _gather_call(
        table_wide, rand_id.astype(jnp.int32), b_per_w, info.num_cores
    )


# speedup vs baseline: 24.2605x; 2.7036x over previous
import functools

import jax
import jax.numpy as jnp
from jax import lax
from jax.experimental import pallas as pl
from jax.experimental.pallas import tpu as pltpu
from jax.experimental.pallas import tpu_sc as plsc

_NSEM = 4


def _gather_call(table, idx, b_per_w, num_cores):
    B = idx.shape[0]
    D = table.shape[1]
    mesh = plsc.VectorSubcoreMesh(core_axis_name="c", subcore_axis_name="s")

    @functools.partial(
        pl.kernel,
        mesh=mesh,
        out_type=jax.ShapeDtypeStruct((B, D), jnp.float32),
        scratch_types=[
            pltpu.VMEM((b_per_w,), jnp.int32),
            pltpu.VMEM((b_per_w, D), jnp.float32),
            [pltpu.SemaphoreType.DMA] * _NSEM,
        ],
    )
    def gather_kernel(table_hbm, idx_hbm, out_hbm, idx_v, rows_v, sems):
        wid = lax.axis_index("s") * num_cores + lax.axis_index("c")
        base = wid * b_per_w
        pltpu.sync_copy(idx_hbm.at[pl.ds(base, b_per_w)], idx_v)

        def fire(g, carry):
            vec = idx_v[pl.ds(g * 16, 16)]
            for lane in range(16):
                row = vec[lane]
                pltpu.make_async_copy(
                    table_hbm.at[pl.ds(row, 1)],
                    rows_v.at[pl.ds(g * 16 + lane, 1)],
                    sems[lane % _NSEM],
                ).start()
            return carry

        lax.fori_loop(0, b_per_w // 16, fire, 0)
        rows_per_sem = b_per_w // _NSEM
        for s in range(_NSEM):
            pltpu.make_async_copy(
                table_hbm.at[pl.ds(0, rows_per_sem)],
                rows_v.at[pl.ds(s * rows_per_sem, rows_per_sem)],
                sems[s],
            ).wait()
        pltpu.sync_copy(rows_v, out_hbm.at[pl.ds(base, b_per_w)])

    return gather_kernel(table, idx)


def kernel(expression_face, rand_id):
    info = plsc.get_sparse_core_info()
    nw = info.num_cores * info.num_subcores
    B = rand_id.shape[0]
    b_per_w = B // nw
    return _gather_call(
        expression_face, rand_id.astype(jnp.int32), b_per_w, info.num_cores
    )
